# Initial kernel scaffold; baseline (speedup 1.0000x reference)
import jax
import jax.numpy as jnp
from jax.experimental import pallas as pl


def kernel(x, atom_origin_type, batch):
    raise NotImplementedError("write your pallas kernel here")


# speedup vs baseline: 1.5375x; 1.5375x over previous
import jax
import jax.numpy as jnp
from jax import lax
from jax.experimental import pallas as pl
from jax.experimental.pallas import tpu as pltpu
from jax.experimental.pallas import tpu_sc as plsc

N_NODES = 100000
D_FEAT = 256
N_GRAPHS = 512
N_CORES = 2
N_SUBCORES = 16
COLS = D_FEAT // N_CORES
ROWS_PER_T = N_NODES // N_SUBCORES
CHUNK = 240
LAST_CHUNK = 250
N_FULL_CHUNKS = 25
LANES = 16
KVECS = COLS // LANES
OUT_STRIPE = N_GRAPHS // N_SUBCORES
IDXBUF = ROWS_PER_T + 14
STAGE = ROWS_PER_T + 6


def _sc_body(x_hbm, type_hbm, batch_hbm, out_hbm, part_hbm,
             type_v, batch_v, xbuf, accum, tmp, rbuf):
    c = lax.axis_index("c")
    s = lax.axis_index("s")
    rbase = s * ROWS_PER_T
    shift = lax.rem(rbase, 8)
    abase = pl.multiple_of(rbase - shift, 8)
    cbase = pl.multiple_of(c * COLS, COLS)

    zero16 = jnp.zeros((LANES,), jnp.float32)

    def zacc(r, carry):
        for k in range(KVECS):
            accum[r, pl.ds(k * LANES, LANES)] = zero16
        return carry

    lax.fori_loop(0, N_GRAPHS, zacc, 0)

    pltpu.sync_copy(type_hbm.at[pl.ds(abase, STAGE)],
                    type_v.at[pl.ds(0, STAGE)])
    pltpu.sync_copy(batch_hbm.at[pl.ds(abase, STAGE)],
                    batch_v.at[pl.ds(0, STAGE)])

    def rows16(chunk_off, gi, n_lanes, carry):
        prev, *acc = carry
        o = chunk_off + gi * LANES + shift
        seg16 = batch_v[pl.ds(o, LANES)]
        t16 = type_v[pl.ds(o, LANES)]
        for r2 in range(n_lanes):
            seg = seg16[r2]
            ok = t16[r2] == 0
            same = seg == prev
            new_acc = []
            for k in range(KVECS):
                a = jnp.where(same, acc[k], zero16)
                xv = xbuf[gi * LANES + r2 + shift, pl.ds(k * LANES, LANES)]
                a = a + jnp.where(ok, xv, zero16)
                accum[seg, pl.ds(k * LANES, LANES)] = a
                new_acc.append(a)
            acc = new_acc
            prev = seg
        return (prev, *acc)

    carry0 = (jnp.int32(-1),) + (zero16,) * KVECS

    def chunk_body(j, carry):
        chunk_off = j * CHUNK

        @pl.when(j < N_FULL_CHUNKS)
        def _():
            pltpu.sync_copy(
                x_hbm.at[pl.ds(abase + chunk_off, CHUNK + 8),
                         pl.ds(cbase, COLS)],
                xbuf.at[pl.ds(0, CHUNK + 8)])

        @pl.when(j == N_FULL_CHUNKS)
        def _():
            pltpu.sync_copy(
                x_hbm.at[pl.ds(abase + chunk_off, LAST_CHUNK + 6),
                         pl.ds(cbase, COLS)],
                xbuf.at[pl.ds(0, LAST_CHUNK + 6)])

        def group_body(gi, carry):
            return rows16(chunk_off, gi, LANES, carry)

        return lax.fori_loop(0, CHUNK // LANES, group_body, carry)

    carry = lax.fori_loop(0, N_FULL_CHUNKS + 1, chunk_body, carry0)
    carry = rows16(N_FULL_CHUNKS * CHUNK, CHUNK // LANES, 10, carry)

    pltpu.sync_copy(accum, part_hbm.at[c].at[s])
    plsc.subcore_barrier()

    def zr(r, carry):
        for k in range(KVECS):
            rbuf[r, pl.ds(k * LANES, LANES)] = zero16
        return carry

    lax.fori_loop(0, OUT_STRIPE, zr, 0)

    def tsum(t, carry):
        pltpu.sync_copy(
            part_hbm.at[c].at[t].at[pl.ds(s * OUT_STRIPE, OUT_STRIPE)], tmp)

        def radd(r, carry2):
            for k in range(KVECS):
                sl = pl.ds(k * LANES, LANES)
                rbuf[r, sl] = rbuf[r, sl] + tmp[r, sl]
            return carry2

        lax.fori_loop(0, OUT_STRIPE, radd, 0)
        return carry

    lax.fori_loop(0, N_SUBCORES, tsum, 0)

    pltpu.sync_copy(rbuf, out_hbm.at[pl.ds(s * OUT_STRIPE, OUT_STRIPE),
                                     pl.ds(cbase, COLS)])


_mesh = plsc.VectorSubcoreMesh(core_axis_name="c", subcore_axis_name="s",
                               num_cores=N_CORES, num_subcores=N_SUBCORES)

_sc_call = pl.kernel(
    _sc_body,
    out_type=(jax.ShapeDtypeStruct((N_GRAPHS, D_FEAT), jnp.float32),
              jax.ShapeDtypeStruct((N_CORES, N_SUBCORES, N_GRAPHS, COLS),
                                   jnp.float32)),
    mesh=_mesh,
    scratch_types=[
        pltpu.VMEM((IDXBUF,), jnp.int32),
        pltpu.VMEM((IDXBUF,), jnp.int32),
        pltpu.VMEM((LAST_CHUNK + 8, COLS), jnp.float32),
        pltpu.VMEM((N_GRAPHS, COLS), jnp.float32),
        pltpu.VMEM((OUT_STRIPE, COLS), jnp.float32),
        pltpu.VMEM((OUT_STRIPE, COLS), jnp.float32),
    ],
)


@jax.jit
def kernel(x, atom_origin_type, batch):
    t = atom_origin_type.astype(jnp.int32)
    b = batch.astype(jnp.int32)
    out, _ = _sc_call(x, t, b)
    return out


# speedup vs baseline: 3.0299x; 1.9707x over previous
import jax
import jax.numpy as jnp
from jax import lax
from jax.experimental import pallas as pl
from jax.experimental.pallas import tpu as pltpu
from jax.experimental.pallas import tpu_sc as plsc

N_NODES = 100000
D_FEAT = 256
N_GRAPHS = 512
N_CORES = 2
N_SUBCORES = 16
COLS = D_FEAT // N_CORES
ROWS_PER_T = N_NODES // N_SUBCORES
LANES = 16
KVECS = COLS // LANES
N_GROUPS = (ROWS_PER_T + LANES - 1) // LANES
CHUNK = 128
TRASH = N_GRAPHS
ACC_ROWS = 520
OUT_STRIPE = N_GRAPHS // N_SUBCORES
STAGE = ROWS_PER_T + 6
IDXBUF = STAGE + 10
CBUF = ROWS_PER_T + CHUNK + 22


def _sc_body(x_hbm, type_hbm, batch_hbm, out_hbm, part_hbm,
             type_v, batch_v, crow, cseg, xbuf, accum, tmp, rbuf):
    c = lax.axis_index("c")
    s = lax.axis_index("s")
    rbase = s * ROWS_PER_T
    shift = lax.rem(rbase, 8)
    abase = pl.multiple_of(rbase - shift, 8)
    cbase = pl.multiple_of(c * COLS, COLS)

    zero16 = jnp.zeros((LANES,), jnp.float32)
    iota16 = lax.iota(jnp.int32, LANES)

    def zacc(r, carry):
        for k in range(KVECS):
            accum[r, pl.ds(k * LANES, LANES)] = zero16
        return carry

    lax.fori_loop(0, ACC_ROWS, zacc, 0)

    pltpu.sync_copy(type_hbm.at[pl.ds(abase, STAGE)],
                    type_v.at[pl.ds(0, STAGE)])
    pltpu.sync_copy(batch_hbm.at[pl.ds(abase, STAGE)],
                    batch_v.at[pl.ds(0, STAGE)])

    def cgroup(gi, cnt):
        o = gi * LANES + shift
        t16 = type_v[pl.ds(o, LANES)]
        seg16 = batch_v[pl.ds(o, LANES)]
        nvalid = jnp.minimum(ROWS_PER_T - gi * LANES, LANES)
        mask = jnp.logical_and(t16 == 0, iota16 < nvalid)
        rid16 = rbase + gi * LANES + iota16
        mi = mask.astype(jnp.int32)
        incl = jnp.cumsum(mi)
        pos = jnp.where(mask, cnt + incl - mi, CBUF - 1)
        plsc.store_scatter(crow, [pos], rid16)
        plsc.store_scatter(cseg, [pos], seg16)
        return cnt + incl[LANES - 1]

    cnt = lax.fori_loop(0, N_GROUPS, cgroup, jnp.int32(0))

    for k in range(CHUNK // LANES):
        crow[pl.ds(cnt + k * LANES, LANES)] = jnp.zeros((LANES,), jnp.int32)
        cseg[pl.ds(cnt + k * LANES, LANES)] = jnp.full((LANES,), TRASH,
                                                       jnp.int32)

    n_chunks = (cnt + CHUNK - 1) // CHUNK

    def chunk_body(ci, carry):
        coff = pl.multiple_of(ci * CHUNK, CHUNK)
        pltpu.sync_copy(
            x_hbm.at[crow.at[pl.ds(coff, CHUNK)], pl.ds(cbase, COLS)],
            xbuf)

        def group_body(gi, carry):
            prev, *acc = carry
            seg16 = cseg[pl.ds(coff + gi * LANES, LANES)]
            for r2 in range(LANES):
                seg = seg16[r2]
                same = seg == prev
                new_acc = []
                for k in range(KVECS):
                    a = jnp.where(same, acc[k], zero16)
                    a = a + xbuf[gi * LANES + r2, pl.ds(k * LANES, LANES)]
                    accum[seg, pl.ds(k * LANES, LANES)] = a
                    new_acc.append(a)
                acc = new_acc
                prev = seg
            return (prev, *acc)

        return lax.fori_loop(0, CHUNK // LANES, group_body, carry)

    carry0 = (jnp.int32(-1),) + (zero16,) * KVECS
    lax.fori_loop(0, n_chunks, chunk_body, carry0)

    pltpu.sync_copy(accum.at[pl.ds(0, N_GRAPHS)], part_hbm.at[c].at[s])
    plsc.subcore_barrier()

    def zr(r, carry):
        for k in range(KVECS):
            rbuf[r, pl.ds(k * LANES, LANES)] = zero16
        return carry

    lax.fori_loop(0, OUT_STRIPE, zr, 0)

    def tsum(t, carry):
        pltpu.sync_copy(
            part_hbm.at[c].at[t].at[pl.ds(s * OUT_STRIPE, OUT_STRIPE)], tmp)

        def radd(r, carry2):
            for k in range(KVECS):
                sl = pl.ds(k * LANES, LANES)
                rbuf[r, sl] = rbuf[r, sl] + tmp[r, sl]
            return carry2

        lax.fori_loop(0, OUT_STRIPE, radd, 0)
        return carry

    lax.fori_loop(0, N_SUBCORES, tsum, 0)

    pltpu.sync_copy(rbuf, out_hbm.at[pl.ds(s * OUT_STRIPE, OUT_STRIPE),
                                     pl.ds(cbase, COLS)])


_mesh = plsc.VectorSubcoreMesh(core_axis_name="c", subcore_axis_name="s",
                               num_cores=N_CORES, num_subcores=N_SUBCORES)

_sc_call = pl.kernel(
    _sc_body,
    out_type=(jax.ShapeDtypeStruct((N_GRAPHS, D_FEAT), jnp.float32),
              jax.ShapeDtypeStruct((N_CORES, N_SUBCORES, N_GRAPHS, COLS),
                                   jnp.float32)),
    mesh=_mesh,
    compiler_params=pltpu.CompilerParams(needs_layout_passes=False),
    scratch_types=[
        pltpu.VMEM((IDXBUF,), jnp.int32),
        pltpu.VMEM((IDXBUF,), jnp.int32),
        pltpu.VMEM((CBUF,), jnp.int32),
        pltpu.VMEM((CBUF,), jnp.int32),
        pltpu.VMEM((CHUNK, COLS), jnp.float32),
        pltpu.VMEM((ACC_ROWS, COLS), jnp.float32),
        pltpu.VMEM((OUT_STRIPE, COLS), jnp.float32),
        pltpu.VMEM((OUT_STRIPE, COLS), jnp.float32),
    ],
)


@jax.jit
def kernel(x, atom_origin_type, batch):
    t = atom_origin_type.astype(jnp.int32)
    b = batch.astype(jnp.int32)
    out, _ = _sc_call(x, t, b)
    return out


# speedup vs baseline: 3.6057x; 1.1900x over previous
import jax
import jax.numpy as jnp
from jax import lax
from jax.experimental import pallas as pl
from jax.experimental.pallas import tpu as pltpu
from jax.experimental.pallas import tpu_sc as plsc

N_NODES = 100000
D_FEAT = 256
N_GRAPHS = 512
N_CORES = 2
N_SUBCORES = 16
COLS = D_FEAT // N_CORES
ROWS_PER_T = N_NODES // N_SUBCORES
LANES = 16
KVECS = COLS // LANES
N_GROUPS = (ROWS_PER_T + LANES - 1) // LANES
CHUNK = 128
TRASH = N_GRAPHS
ACC_ROWS = 520
OUT_STRIPE = N_GRAPHS // N_SUBCORES
STAGE = ROWS_PER_T + 6
IDXBUF = STAGE + 10
CBUF = ROWS_PER_T + CHUNK + 22


def _sc_body(x_hbm, type_hbm, batch_hbm, out_hbm, part_hbm,
             type_v, batch_v, crow, cseg, xbuf, accum, tmp, rbuf):
    c = lax.axis_index("c")
    s = lax.axis_index("s")
    rbase = s * ROWS_PER_T
    shift = lax.rem(rbase, 8)
    abase = pl.multiple_of(rbase - shift, 8)
    cbase = pl.multiple_of(c * COLS, COLS)

    zero16 = jnp.zeros((LANES,), jnp.float32)
    iota16 = lax.iota(jnp.int32, LANES)

    def zacc(r, carry):
        for k in range(KVECS):
            accum[r, pl.ds(k * LANES, LANES)] = zero16
        return carry

    lax.fori_loop(0, ACC_ROWS, zacc, 0)

    pltpu.sync_copy(type_hbm.at[pl.ds(abase, STAGE)],
                    type_v.at[pl.ds(0, STAGE)])
    pltpu.sync_copy(batch_hbm.at[pl.ds(abase, STAGE)],
                    batch_v.at[pl.ds(0, STAGE)])

    def cgroup(gi, cnt):
        o = gi * LANES + shift
        t16 = type_v[pl.ds(o, LANES)]
        seg16 = batch_v[pl.ds(o, LANES)]
        nvalid = jnp.minimum(ROWS_PER_T - gi * LANES, LANES)
        mask = jnp.logical_and(t16 == 0, iota16 < nvalid)
        rid16 = rbase + gi * LANES + iota16
        mi = mask.astype(jnp.int32)
        incl = jnp.cumsum(mi)
        pos = jnp.where(mask, cnt + incl - mi, CBUF - 1)
        plsc.store_scatter(crow, [pos], rid16)
        plsc.store_scatter(cseg, [pos], seg16)
        return cnt + incl[LANES - 1]

    cnt = lax.fori_loop(0, N_GROUPS, cgroup, jnp.int32(0))

    for k in range(CHUNK // LANES):
        crow[pl.ds(cnt + k * LANES, LANES)] = jnp.zeros((LANES,), jnp.int32)
        cseg[pl.ds(cnt + k * LANES, LANES)] = jnp.full((LANES,), TRASH,
                                                       jnp.int32)

    n_chunks = (cnt + CHUNK - 1) // CHUNK

    def chunk_body(ci, carry):
        coff = pl.multiple_of(ci * CHUNK, CHUNK)
        pltpu.sync_copy(
            x_hbm.at[crow.at[pl.ds(coff, CHUNK)], pl.ds(cbase, COLS)],
            xbuf)

        def group_body(gi, carry):
            prev = carry[0]
            seg16 = cseg[pl.ds(coff + gi * LANES, LANES)]
            one_run = jnp.all(seg16 == prev)

            def fast(carry):
                prev, *acc = carry
                for r2 in range(LANES):
                    acc = [acc[k] + xbuf[gi * LANES + r2,
                                         pl.ds(k * LANES, LANES)]
                           for k in range(KVECS)]
                return (prev, *acc)

            def slow(carry):
                prev, *acc = carry
                for r2 in range(LANES):
                    seg = seg16[r2]
                    same = seg == prev
                    new_acc = []
                    for k in range(KVECS):
                        accum[prev, pl.ds(k * LANES, LANES)] = acc[k]
                        a = jnp.where(same, acc[k], zero16)
                        a = a + xbuf[gi * LANES + r2, pl.ds(k * LANES, LANES)]
                        accum[seg, pl.ds(k * LANES, LANES)] = a
                        new_acc.append(a)
                    acc = new_acc
                    prev = seg
                return (prev, *acc)

            return lax.cond(one_run, fast, slow, carry)

        return lax.fori_loop(0, CHUNK // LANES, group_body, carry)

    carry0 = (jnp.int32(TRASH),) + (zero16,) * KVECS
    fprev, *facc = lax.fori_loop(0, n_chunks, chunk_body, carry0)
    for k in range(KVECS):
        accum[fprev, pl.ds(k * LANES, LANES)] = facc[k]

    pltpu.sync_copy(accum.at[pl.ds(0, N_GRAPHS)], part_hbm.at[c].at[s])
    plsc.subcore_barrier()

    def zr(r, carry):
        for k in range(KVECS):
            rbuf[r, pl.ds(k * LANES, LANES)] = zero16
        return carry

    lax.fori_loop(0, OUT_STRIPE, zr, 0)

    def tsum(t, carry):
        pltpu.sync_copy(
            part_hbm.at[c].at[t].at[pl.ds(s * OUT_STRIPE, OUT_STRIPE)], tmp)

        def radd(r, carry2):
            for k in range(KVECS):
                sl = pl.ds(k * LANES, LANES)
                rbuf[r, sl] = rbuf[r, sl] + tmp[r, sl]
            return carry2

        lax.fori_loop(0, OUT_STRIPE, radd, 0)
        return carry

    lax.fori_loop(0, N_SUBCORES, tsum, 0)

    pltpu.sync_copy(rbuf, out_hbm.at[pl.ds(s * OUT_STRIPE, OUT_STRIPE),
                                     pl.ds(cbase, COLS)])


_mesh = plsc.VectorSubcoreMesh(core_axis_name="c", subcore_axis_name="s",
                               num_cores=N_CORES, num_subcores=N_SUBCORES)

_sc_call = pl.kernel(
    _sc_body,
    out_type=(jax.ShapeDtypeStruct((N_GRAPHS, D_FEAT), jnp.float32),
              jax.ShapeDtypeStruct((N_CORES, N_SUBCORES, N_GRAPHS, COLS),
                                   jnp.float32)),
    mesh=_mesh,
    compiler_params=pltpu.CompilerParams(needs_layout_passes=False),
    scratch_types=[
        pltpu.VMEM((IDXBUF,), jnp.int32),
        pltpu.VMEM((IDXBUF,), jnp.int32),
        pltpu.VMEM((CBUF,), jnp.int32),
        pltpu.VMEM((CBUF,), jnp.int32),
        pltpu.VMEM((CHUNK, COLS), jnp.float32),
        pltpu.VMEM((ACC_ROWS, COLS), jnp.float32),
        pltpu.VMEM((OUT_STRIPE, COLS), jnp.float32),
        pltpu.VMEM((OUT_STRIPE, COLS), jnp.float32),
    ],
)


@jax.jit
def kernel(x, atom_origin_type, batch):
    t = atom_origin_type.astype(jnp.int32)
    b = batch.astype(jnp.int32)
    out, _ = _sc_call(x, t, b)
    return out


# speedup vs baseline: 4.1193x; 1.1424x over previous
import jax
import jax.numpy as jnp
from jax import lax
from jax.experimental import pallas as pl
from jax.experimental.pallas import tpu as pltpu
from jax.experimental.pallas import tpu_sc as plsc

N_NODES = 100000
D_FEAT = 256
N_GRAPHS = 512
N_CORES = 2
N_SUBCORES = 16
COLS = D_FEAT // N_CORES
ROWS_PER_T = N_NODES // N_SUBCORES
LANES = 16
KVECS = COLS // LANES
N_GROUPS = (ROWS_PER_T + LANES - 1) // LANES
CHUNK = 96
TRASH = N_GRAPHS
ACC_ROWS = 513
OUT_STRIPE = N_GRAPHS // N_SUBCORES
STAGE = ROWS_PER_T + 6
IDXBUF = STAGE + 10
CBUF = ROWS_PER_T + CHUNK + 22


def _sc_body(x_hbm, type_hbm, batch_hbm, out_hbm, part_hbm,
             type_v, batch_v, crow, cseg, xbuf, accum, tmp, rbuf,
             gsem0, gsem1, rsem0, rsem1):
    c = lax.axis_index("c")
    s = lax.axis_index("s")
    rbase = s * ROWS_PER_T
    shift = lax.rem(rbase, 8)
    abase = pl.multiple_of(rbase - shift, 8)
    cbase = pl.multiple_of(c * COLS, COLS)

    zero16 = jnp.zeros((LANES,), jnp.float32)
    iota16 = lax.iota(jnp.int32, LANES)

    def zacc(r, carry):
        for k in range(KVECS):
            accum[r, pl.ds(k * LANES, LANES)] = zero16
        return carry

    lax.fori_loop(0, ACC_ROWS, zacc, 0)

    pltpu.sync_copy(type_hbm.at[pl.ds(abase, STAGE)],
                    type_v.at[pl.ds(0, STAGE)])
    pltpu.sync_copy(batch_hbm.at[pl.ds(abase, STAGE)],
                    batch_v.at[pl.ds(0, STAGE)])

    def cgroup(gi, cnt):
        o = gi * LANES + shift
        t16 = type_v[pl.ds(o, LANES)]
        seg16 = batch_v[pl.ds(o, LANES)]
        nvalid = jnp.minimum(ROWS_PER_T - gi * LANES, LANES)
        mask = jnp.logical_and(t16 == 0, iota16 < nvalid)
        rid16 = rbase + gi * LANES + iota16
        mi = mask.astype(jnp.int32)
        incl = jnp.cumsum(mi)
        pos = jnp.where(mask, cnt + incl - mi, CBUF - 1)
        plsc.store_scatter(crow, [pos], rid16)
        plsc.store_scatter(cseg, [pos], seg16)
        return cnt + incl[LANES - 1]

    cnt = lax.fori_loop(0, N_GROUPS, cgroup, jnp.int32(0))

    for k in range(CHUNK // LANES):
        crow[pl.ds(cnt + k * LANES, LANES)] = jnp.zeros((LANES,), jnp.int32)
        cseg[pl.ds(cnt + k * LANES, LANES)] = jnp.full((LANES,), TRASH,
                                                       jnp.int32)

    n_chunks = (cnt + CHUNK - 1) // CHUNK

    def gather(coff, buf, sem):
        pltpu.async_copy(
            x_hbm.at[crow.at[pl.ds(coff, CHUNK)], pl.ds(cbase, COLS)],
            buf, sem)

    def gwait(buf, sem):
        pltpu.make_async_copy(
            x_hbm.at[crow.at[pl.ds(0, CHUNK)], pl.ds(cbase, COLS)],
            buf, sem).wait()

    @pl.when(n_chunks > 0)
    def _():
        gather(pl.multiple_of(jnp.int32(0), 8), xbuf.at[0], gsem0)

    def chunk_body(ci, carry):
        par = lax.rem(ci, 2)
        noff = pl.multiple_of((ci + 1) * CHUNK, 8)

        @pl.when(par == 0)
        def _():
            gwait(xbuf.at[0], gsem0)

            @pl.when(ci + 1 < n_chunks)
            def _():
                gather(noff, xbuf.at[1], gsem1)

        @pl.when(par == 1)
        def _():
            gwait(xbuf.at[1], gsem1)

            @pl.when(ci + 1 < n_chunks)
            def _():
                gather(noff, xbuf.at[0], gsem0)

        coff = pl.multiple_of(ci * CHUNK, 8)

        def group_body(gi, carry):
            prev = carry[0]
            seg16 = cseg[pl.ds(coff + gi * LANES, LANES)]
            one_run = jnp.all(seg16 == prev)

            def fast(carry):
                prev, *acc = carry
                for r2 in range(LANES):
                    acc = [acc[k] + xbuf[par, gi * LANES + r2,
                                         pl.ds(k * LANES, LANES)]
                           for k in range(KVECS)]
                return (prev, *acc)

            def slow(carry):
                prev, *acc = carry
                for r2 in range(LANES):
                    seg = seg16[r2]
                    same = seg == prev
                    new_acc = []
                    for k in range(KVECS):
                        accum[prev, pl.ds(k * LANES, LANES)] = acc[k]
                        a = jnp.where(same, acc[k], zero16)
                        a = a + xbuf[par, gi * LANES + r2,
                                     pl.ds(k * LANES, LANES)]
                        accum[seg, pl.ds(k * LANES, LANES)] = a
                        new_acc.append(a)
                    acc = new_acc
                    prev = seg
                return (prev, *acc)

            return lax.cond(one_run, fast, slow, carry)

        return lax.fori_loop(0, CHUNK // LANES, group_body, carry)

    carry0 = (jnp.int32(TRASH),) + (zero16,) * KVECS
    fprev, *facc = lax.fori_loop(0, n_chunks, chunk_body, carry0)
    for k in range(KVECS):
        accum[fprev, pl.ds(k * LANES, LANES)] = facc[k]

    pltpu.sync_copy(accum.at[pl.ds(0, N_GRAPHS)], part_hbm.at[c].at[s])
    plsc.subcore_barrier()

    def stripe(t):
        return part_hbm.at[c].at[t].at[pl.ds(s * OUT_STRIPE, OUT_STRIPE)]

    pltpu.async_copy(stripe(jnp.int32(0)), tmp.at[0], rsem0)

    def tsum(t, carry):
        par = lax.rem(t, 2)

        @pl.when(par == 0)
        def _():
            pltpu.make_async_copy(stripe(t), tmp.at[0], rsem0).wait()

            @pl.when(t + 1 < N_SUBCORES)
            def _():
                pltpu.async_copy(stripe(t + 1), tmp.at[1], rsem1)

        @pl.when(par == 1)
        def _():
            pltpu.make_async_copy(stripe(t), tmp.at[1], rsem1).wait()

            @pl.when(t + 1 < N_SUBCORES)
            def _():
                pltpu.async_copy(stripe(t + 1), tmp.at[0], rsem0)

        def radd(r, carry2):
            for k in range(KVECS):
                sl = pl.ds(k * LANES, LANES)
                v = tmp[par, r, sl]
                rbuf[r, sl] = jnp.where(t == 0, v, rbuf[r, sl] + v)
            return carry2

        lax.fori_loop(0, OUT_STRIPE, radd, 0)
        return carry

    lax.fori_loop(0, N_SUBCORES, tsum, 0)

    pltpu.sync_copy(rbuf, out_hbm.at[pl.ds(s * OUT_STRIPE, OUT_STRIPE),
                                     pl.ds(cbase, COLS)])


_mesh = plsc.VectorSubcoreMesh(core_axis_name="c", subcore_axis_name="s",
                               num_cores=N_CORES, num_subcores=N_SUBCORES)

_sc_call = pl.kernel(
    _sc_body,
    out_type=(jax.ShapeDtypeStruct((N_GRAPHS, D_FEAT), jnp.float32),
              jax.ShapeDtypeStruct((N_CORES, N_SUBCORES, N_GRAPHS, COLS),
                                   jnp.float32)),
    mesh=_mesh,
    compiler_params=pltpu.CompilerParams(needs_layout_passes=False),
    scratch_types=[
        pltpu.VMEM((IDXBUF,), jnp.int32),
        pltpu.VMEM((IDXBUF,), jnp.int32),
        pltpu.VMEM((CBUF,), jnp.int32),
        pltpu.VMEM((CBUF,), jnp.int32),
        pltpu.VMEM((2, CHUNK, COLS), jnp.float32),
        pltpu.VMEM((ACC_ROWS, COLS), jnp.float32),
        pltpu.VMEM((2, OUT_STRIPE, COLS), jnp.float32),
        pltpu.VMEM((OUT_STRIPE, COLS), jnp.float32),
        pltpu.SemaphoreType.DMA,
        pltpu.SemaphoreType.DMA,
        pltpu.SemaphoreType.DMA,
        pltpu.SemaphoreType.DMA,
    ],
)


@jax.jit
def kernel(x, atom_origin_type, batch):
    t = atom_origin_type.astype(jnp.int32)
    b = batch.astype(jnp.int32)
    out, _ = _sc_call(x, t, b)
    return out


# speedup vs baseline: 4.1253x; 1.0015x over previous
import jax
import jax.numpy as jnp
from jax import lax
from jax.experimental import pallas as pl
from jax.experimental.pallas import tpu as pltpu
from jax.experimental.pallas import tpu_sc as plsc

N_NODES = 100000
D_FEAT = 256
N_GRAPHS = 512
N_CORES = 2
N_SUBCORES = 16
COLS = D_FEAT // N_CORES
ROWS_PER_T = N_NODES // N_SUBCORES
LANES = 16
KVECS = COLS // LANES
N_GROUPS = (ROWS_PER_T + LANES - 1) // LANES
CHUNK = 96
TRASH = N_GRAPHS
ACC_ROWS = 513
OUT_STRIPE = N_GRAPHS // N_SUBCORES
STAGE = ROWS_PER_T + 6
IDXBUF = STAGE + 10
CBUF = ROWS_PER_T + CHUNK + 22


def _sc_body(x_hbm, type_hbm, batch_hbm, out_hbm, part_hbm,
             type_v, batch_v, crow, cseg, xbuf, accum, tmp, rbuf,
             gsem0, gsem1, rsem0, rsem1):
    c = lax.axis_index("c")
    s = lax.axis_index("s")
    rbase = s * ROWS_PER_T
    shift = lax.rem(rbase, 8)
    abase = pl.multiple_of(rbase - shift, 8)
    cbase = pl.multiple_of(c * COLS, COLS)

    zero16 = jnp.zeros((LANES,), jnp.float32)
    iota16 = lax.iota(jnp.int32, LANES)

    scope_zero = jax.named_scope("ph_zero")
    scope_zero.__enter__()
    def zacc(r, carry):
        for k in range(KVECS):
            accum[r, pl.ds(k * LANES, LANES)] = zero16
        return carry

    lax.fori_loop(0, ACC_ROWS, zacc, 0)
    scope_zero.__exit__(None, None, None)

    pltpu.sync_copy(type_hbm.at[pl.ds(abase, STAGE)],
                    type_v.at[pl.ds(0, STAGE)])
    pltpu.sync_copy(batch_hbm.at[pl.ds(abase, STAGE)],
                    batch_v.at[pl.ds(0, STAGE)])

    def cgroup(gi, cnt):
        o = gi * LANES + shift
        t16 = type_v[pl.ds(o, LANES)]
        seg16 = batch_v[pl.ds(o, LANES)]
        nvalid = jnp.minimum(ROWS_PER_T - gi * LANES, LANES)
        mask = jnp.logical_and(t16 == 0, iota16 < nvalid)
        rid16 = rbase + gi * LANES + iota16
        mi = mask.astype(jnp.int32)
        incl = jnp.cumsum(mi)
        pos = jnp.where(mask, cnt + incl - mi, CBUF - 1)
        plsc.store_scatter(crow, [pos], rid16)
        plsc.store_scatter(cseg, [pos], seg16)
        return cnt + incl[LANES - 1]

    with jax.named_scope("ph_compact"):
        cnt = lax.fori_loop(0, N_GROUPS, cgroup, jnp.int32(0))

    for k in range(CHUNK // LANES):
        crow[pl.ds(cnt + k * LANES, LANES)] = jnp.zeros((LANES,), jnp.int32)
        cseg[pl.ds(cnt + k * LANES, LANES)] = jnp.full((LANES,), TRASH,
                                                       jnp.int32)

    n_chunks = (cnt + CHUNK - 1) // CHUNK

    def gather(coff, buf, sem):
        pltpu.async_copy(
            x_hbm.at[crow.at[pl.ds(coff, CHUNK)], pl.ds(cbase, COLS)],
            buf, sem)

    def gwait(buf, sem):
        pltpu.make_async_copy(
            x_hbm.at[crow.at[pl.ds(0, CHUNK)], pl.ds(cbase, COLS)],
            buf, sem).wait()

    @pl.when(n_chunks > 0)
    def _():
        gather(pl.multiple_of(jnp.int32(0), 8), xbuf.at[0], gsem0)

    def chunk_body(ci, carry):
        par = lax.rem(ci, 2)
        noff = pl.multiple_of((ci + 1) * CHUNK, 8)

        @pl.when(par == 0)
        def _():
            gwait(xbuf.at[0], gsem0)

            @pl.when(ci + 1 < n_chunks)
            def _():
                gather(noff, xbuf.at[1], gsem1)

        @pl.when(par == 1)
        def _():
            gwait(xbuf.at[1], gsem1)

            @pl.when(ci + 1 < n_chunks)
            def _():
                gather(noff, xbuf.at[0], gsem0)

        coff = pl.multiple_of(ci * CHUNK, 8)

        def group_body(gi, carry):
            prev = carry[0]
            seg16 = cseg[pl.ds(coff + gi * LANES, LANES)]
            one_run = jnp.all(seg16 == prev)

            def fast(carry):
                prev, *acc = carry
                for r2 in range(LANES):
                    acc = [acc[k] + xbuf[par, gi * LANES + r2,
                                         pl.ds(k * LANES, LANES)]
                           for k in range(KVECS)]
                return (prev, *acc)

            def slow(carry):
                prev, *acc = carry
                for r2 in range(LANES):
                    seg = seg16[r2]
                    same = seg == prev
                    new_acc = []
                    for k in range(KVECS):
                        accum[prev, pl.ds(k * LANES, LANES)] = acc[k]
                        a = jnp.where(same, acc[k], zero16)
                        a = a + xbuf[par, gi * LANES + r2,
                                     pl.ds(k * LANES, LANES)]
                        accum[seg, pl.ds(k * LANES, LANES)] = a
                        new_acc.append(a)
                    acc = new_acc
                    prev = seg
                return (prev, *acc)

            return lax.cond(one_run, fast, slow, carry)

        return lax.fori_loop(0, CHUNK // LANES, group_body, carry)

    carry0 = (jnp.int32(TRASH),) + (zero16,) * KVECS
    with jax.named_scope("ph_main"):
        fprev, *facc = lax.fori_loop(0, n_chunks, chunk_body, carry0)
    for k in range(KVECS):
        accum[fprev, pl.ds(k * LANES, LANES)] = facc[k]

    with jax.named_scope("ph_partwrite"):
        pltpu.sync_copy(accum.at[pl.ds(0, N_GRAPHS)], part_hbm.at[c].at[s])
    with jax.named_scope("ph_barrier"):
        plsc.subcore_barrier()

    def stripe(t):
        return part_hbm.at[c].at[t].at[pl.ds(s * OUT_STRIPE, OUT_STRIPE)]

    pltpu.async_copy(stripe(jnp.int32(0)), tmp.at[0], rsem0)

    def tsum(t, carry):
        par = lax.rem(t, 2)

        @pl.when(par == 0)
        def _():
            pltpu.make_async_copy(stripe(t), tmp.at[0], rsem0).wait()

            @pl.when(t + 1 < N_SUBCORES)
            def _():
                pltpu.async_copy(stripe(t + 1), tmp.at[1], rsem1)

        @pl.when(par == 1)
        def _():
            pltpu.make_async_copy(stripe(t), tmp.at[1], rsem1).wait()

            @pl.when(t + 1 < N_SUBCORES)
            def _():
                pltpu.async_copy(stripe(t + 1), tmp.at[0], rsem0)

        def radd(r, carry2):
            for k in range(KVECS):
                sl = pl.ds(k * LANES, LANES)
                v = tmp[par, r, sl]
                rbuf[r, sl] = jnp.where(t == 0, v, rbuf[r, sl] + v)
            return carry2

        lax.fori_loop(0, OUT_STRIPE, radd, 0)
        return carry

    with jax.named_scope("ph_reduce"):
        lax.fori_loop(0, N_SUBCORES, tsum, 0)

    pltpu.sync_copy(rbuf, out_hbm.at[pl.ds(s * OUT_STRIPE, OUT_STRIPE),
                                     pl.ds(cbase, COLS)])


_mesh = plsc.VectorSubcoreMesh(core_axis_name="c", subcore_axis_name="s",
                               num_cores=N_CORES, num_subcores=N_SUBCORES)

_sc_call = pl.kernel(
    _sc_body,
    out_type=(jax.ShapeDtypeStruct((N_GRAPHS, D_FEAT), jnp.float32),
              jax.ShapeDtypeStruct((N_CORES, N_SUBCORES, N_GRAPHS, COLS),
                                   jnp.float32)),
    mesh=_mesh,
    compiler_params=pltpu.CompilerParams(needs_layout_passes=False),
    scratch_types=[
        pltpu.VMEM((IDXBUF,), jnp.int32),
        pltpu.VMEM((IDXBUF,), jnp.int32),
        pltpu.VMEM((CBUF,), jnp.int32),
        pltpu.VMEM((CBUF,), jnp.int32),
        pltpu.VMEM((2, CHUNK, COLS), jnp.float32),
        pltpu.VMEM((ACC_ROWS, COLS), jnp.float32),
        pltpu.VMEM((2, OUT_STRIPE, COLS), jnp.float32),
        pltpu.VMEM((OUT_STRIPE, COLS), jnp.float32),
        pltpu.SemaphoreType.DMA,
        pltpu.SemaphoreType.DMA,
        pltpu.SemaphoreType.DMA,
        pltpu.SemaphoreType.DMA,
    ],
)


@jax.jit
def kernel(x, atom_origin_type, batch):
    t = atom_origin_type.astype(jnp.int32)
    b = batch.astype(jnp.int32)
    out, _ = _sc_call(x, t, b)
    return out


# speedup vs baseline: 4.7521x; 1.1519x over previous
import jax
import jax.numpy as jnp
from jax import lax
from jax.experimental import pallas as pl
from jax.experimental.pallas import tpu as pltpu
from jax.experimental.pallas import tpu_sc as plsc

N_NODES = 100000
D_FEAT = 256
N_GRAPHS = 512
N_CORES = 2
N_SUBCORES = 16
COLS = D_FEAT // N_CORES
ROWS_PER_T = N_NODES // N_SUBCORES
LANES = 16
KVECS = COLS // LANES
N_GROUPS = (ROWS_PER_T + LANES - 1) // LANES
CHUNK = 48
NBUF = 4
TRASH = N_GRAPHS
ACC_ROWS = 513
OUT_STRIPE = N_GRAPHS // N_SUBCORES
STAGE = ROWS_PER_T + 6
IDXBUF = STAGE + 10
CBUF = ROWS_PER_T + CHUNK + 22


def _sc_body(x_hbm, type_hbm, batch_hbm, out_hbm, part_hbm,
             type_v, batch_v, crow, cseg, xbuf, accum, tmp, rbuf,
             gsem0, gsem1, gsem2, gsem3, rsem0, rsem1):
    c = lax.axis_index("c")
    s = lax.axis_index("s")
    rbase = s * ROWS_PER_T
    shift = lax.rem(rbase, 8)
    abase = pl.multiple_of(rbase - shift, 8)
    cbase = pl.multiple_of(c * COLS, COLS)

    zero16 = jnp.zeros((LANES,), jnp.float32)
    iota16 = lax.iota(jnp.int32, LANES)

    def zacc(r, carry):
        for k in range(KVECS):
            accum[r, pl.ds(k * LANES, LANES)] = zero16
        return carry

    lax.fori_loop(0, ACC_ROWS, zacc, 0)

    pltpu.sync_copy(type_hbm.at[pl.ds(abase, STAGE)],
                    type_v.at[pl.ds(0, STAGE)])
    pltpu.sync_copy(batch_hbm.at[pl.ds(abase, STAGE)],
                    batch_v.at[pl.ds(0, STAGE)])

    def cgroup(gi, cnt):
        o = gi * LANES + shift
        t16 = type_v[pl.ds(o, LANES)]
        seg16 = batch_v[pl.ds(o, LANES)]
        nvalid = jnp.minimum(ROWS_PER_T - gi * LANES, LANES)
        mask = jnp.logical_and(t16 == 0, iota16 < nvalid)
        rid16 = rbase + gi * LANES + iota16
        mi = mask.astype(jnp.int32)
        incl = jnp.cumsum(mi)
        pos = jnp.where(mask, cnt + incl - mi, CBUF - 1)
        plsc.store_scatter(crow, [pos], rid16)
        plsc.store_scatter(cseg, [pos], seg16)
        return cnt + incl[LANES - 1]

    cnt = lax.fori_loop(0, N_GROUPS, cgroup, jnp.int32(0))

    for k in range(CHUNK // LANES):
        crow[pl.ds(cnt + k * LANES, LANES)] = jnp.zeros((LANES,), jnp.int32)
        cseg[pl.ds(cnt + k * LANES, LANES)] = jnp.full((LANES,), TRASH,
                                                       jnp.int32)

    n_chunks = (cnt + CHUNK - 1) // CHUNK

    def gather(coff, buf, sem):
        pltpu.async_copy(
            x_hbm.at[crow.at[pl.ds(coff, CHUNK)], pl.ds(cbase, COLS)],
            buf, sem)

    def gwait(buf, sem):
        pltpu.make_async_copy(
            x_hbm.at[crow.at[pl.ds(0, CHUNK)], pl.ds(cbase, COLS)],
            buf, sem).wait()

    gsems = (gsem0, gsem1, gsem2, gsem3)
    for b in range(NBUF - 1):
        @pl.when(jnp.int32(b) < n_chunks)
        def _(b=b):
            gather(pl.multiple_of(jnp.int32(b * CHUNK), 8),
                   xbuf.at[b], gsems[b])

    def chunk_body(ci, carry):
        par = lax.rem(ci, NBUF)
        noff = pl.multiple_of((ci + NBUF - 1) * CHUNK, 8)

        for b in range(NBUF):
            @pl.when(par == b)
            def _(b=b):
                gwait(xbuf.at[b], gsems[b])
                nb = (b + NBUF - 1) % NBUF

                @pl.when(ci + NBUF - 1 < n_chunks)
                def _(b=b, nb=nb):
                    gather(noff, xbuf.at[nb], gsems[nb])

        coff = pl.multiple_of(ci * CHUNK, 8)

        def group_body(gi, carry):
            prev = carry[0]
            seg16 = cseg[pl.ds(coff + gi * LANES, LANES)]
            one_run = jnp.all(seg16 == prev)

            def fast(carry):
                prev, *acc = carry
                for r2 in range(LANES):
                    acc = [acc[k] + xbuf[par, gi * LANES + r2,
                                         pl.ds(k * LANES, LANES)]
                           for k in range(KVECS)]
                return (prev, *acc)

            def slow(carry):
                prev, *acc = carry
                for r2 in range(LANES):
                    seg = seg16[r2]
                    same = seg == prev
                    new_acc = []
                    for k in range(KVECS):
                        accum[prev, pl.ds(k * LANES, LANES)] = acc[k]
                        a = jnp.where(same, acc[k], zero16)
                        a = a + xbuf[par, gi * LANES + r2,
                                     pl.ds(k * LANES, LANES)]
                        accum[seg, pl.ds(k * LANES, LANES)] = a
                        new_acc.append(a)
                    acc = new_acc
                    prev = seg
                return (prev, *acc)

            return lax.cond(one_run, fast, slow, carry)

        return lax.fori_loop(0, CHUNK // LANES, group_body, carry)

    carry0 = (jnp.int32(TRASH),) + (zero16,) * KVECS
    fprev, *facc = lax.fori_loop(0, n_chunks, chunk_body, carry0)
    for k in range(KVECS):
        accum[fprev, pl.ds(k * LANES, LANES)] = facc[k]

    pltpu.sync_copy(accum.at[pl.ds(0, N_GRAPHS)], part_hbm.at[c].at[s])
    plsc.subcore_barrier()

    def stripe(t):
        return part_hbm.at[c].at[t].at[pl.ds(s * OUT_STRIPE, OUT_STRIPE)]

    pltpu.async_copy(stripe(jnp.int32(0)), tmp.at[0], rsem0)

    def tsum(t, carry):
        par = lax.rem(t, 2)

        @pl.when(par == 0)
        def _():
            pltpu.make_async_copy(stripe(t), tmp.at[0], rsem0).wait()

            @pl.when(t + 1 < N_SUBCORES)
            def _():
                pltpu.async_copy(stripe(t + 1), tmp.at[1], rsem1)

        @pl.when(par == 1)
        def _():
            pltpu.make_async_copy(stripe(t), tmp.at[1], rsem1).wait()

            @pl.when(t + 1 < N_SUBCORES)
            def _():
                pltpu.async_copy(stripe(t + 1), tmp.at[0], rsem0)

        def radd(r, carry2):
            for k in range(KVECS):
                sl = pl.ds(k * LANES, LANES)
                v = tmp[par, r, sl]
                rbuf[r, sl] = jnp.where(t == 0, v, rbuf[r, sl] + v)
            return carry2

        lax.fori_loop(0, OUT_STRIPE, radd, 0)
        return carry

    lax.fori_loop(0, N_SUBCORES, tsum, 0)

    pltpu.sync_copy(rbuf, out_hbm.at[pl.ds(s * OUT_STRIPE, OUT_STRIPE),
                                     pl.ds(cbase, COLS)])


_mesh = plsc.VectorSubcoreMesh(core_axis_name="c", subcore_axis_name="s",
                               num_cores=N_CORES, num_subcores=N_SUBCORES)

_sc_call = pl.kernel(
    _sc_body,
    out_type=(jax.ShapeDtypeStruct((N_GRAPHS, D_FEAT), jnp.float32),
              jax.ShapeDtypeStruct((N_CORES, N_SUBCORES, N_GRAPHS, COLS),
                                   jnp.float32)),
    mesh=_mesh,
    compiler_params=pltpu.CompilerParams(needs_layout_passes=False),
    scratch_types=[
        pltpu.VMEM((IDXBUF,), jnp.int32),
        pltpu.VMEM((IDXBUF,), jnp.int32),
        pltpu.VMEM((CBUF,), jnp.int32),
        pltpu.VMEM((CBUF,), jnp.int32),
        pltpu.VMEM((NBUF, CHUNK, COLS), jnp.float32),
        pltpu.VMEM((ACC_ROWS, COLS), jnp.float32),
        pltpu.VMEM((2, OUT_STRIPE, COLS), jnp.float32),
        pltpu.VMEM((OUT_STRIPE, COLS), jnp.float32),
        pltpu.SemaphoreType.DMA,
        pltpu.SemaphoreType.DMA,
        pltpu.SemaphoreType.DMA,
        pltpu.SemaphoreType.DMA,
        pltpu.SemaphoreType.DMA,
        pltpu.SemaphoreType.DMA,
    ],
)


@jax.jit
def kernel(x, atom_origin_type, batch):
    t = atom_origin_type.astype(jnp.int32)
    b = batch.astype(jnp.int32)
    out, _ = _sc_call(x, t, b)
    return out


# speedup vs baseline: 5.1047x; 1.0742x over previous
import jax
import jax.numpy as jnp
from jax import lax
from jax.experimental import pallas as pl
from jax.experimental.pallas import tpu as pltpu
from jax.experimental.pallas import tpu_sc as plsc

N_NODES = 100000
D_FEAT = 256
N_GRAPHS = 512
N_CORES = 2
N_SUBCORES = 16
COLS = D_FEAT // N_CORES
ROWS_PER_T = N_NODES // N_SUBCORES
LANES = 16
KVECS = COLS // LANES
N_GROUPS = (ROWS_PER_T + LANES - 1) // LANES
CHUNK = 32
NBUF = 6
TRASH = N_GRAPHS
ACC_ROWS = 513
OUT_STRIPE = N_GRAPHS // N_SUBCORES
STAGE = ROWS_PER_T + 6
IDXBUF = STAGE + 10
CBUF = ROWS_PER_T + CHUNK + 22


def _sc_body(x_hbm, type_hbm, batch_hbm, out_hbm, part_hbm,
             type_v, batch_v, crow, cseg, xbuf, accum, tmp, rbuf,
             gsem0, gsem1, gsem2, gsem3, gsem4, gsem5, rsem0, rsem1):
    c = lax.axis_index("c")
    s = lax.axis_index("s")
    rbase = s * ROWS_PER_T
    shift = lax.rem(rbase, 8)
    abase = pl.multiple_of(rbase - shift, 8)
    cbase = pl.multiple_of(c * COLS, COLS)

    zero16 = jnp.zeros((LANES,), jnp.float32)
    iota16 = lax.iota(jnp.int32, LANES)

    def zacc(r, carry):
        for k in range(KVECS):
            accum[r, pl.ds(k * LANES, LANES)] = zero16
        return carry

    lax.fori_loop(0, ACC_ROWS, zacc, 0)

    pltpu.sync_copy(type_hbm.at[pl.ds(abase, STAGE)],
                    type_v.at[pl.ds(0, STAGE)])
    pltpu.sync_copy(batch_hbm.at[pl.ds(abase, STAGE)],
                    batch_v.at[pl.ds(0, STAGE)])

    def cgroup(gi, cnt):
        o = gi * LANES + shift
        t16 = type_v[pl.ds(o, LANES)]
        seg16 = batch_v[pl.ds(o, LANES)]
        nvalid = jnp.minimum(ROWS_PER_T - gi * LANES, LANES)
        mask = jnp.logical_and(t16 == 0, iota16 < nvalid)
        rid16 = rbase + gi * LANES + iota16
        mi = mask.astype(jnp.int32)
        incl = jnp.cumsum(mi)
        pos = jnp.where(mask, cnt + incl - mi, CBUF - 1)
        plsc.store_scatter(crow, [pos], rid16)
        plsc.store_scatter(cseg, [pos], seg16)
        return cnt + incl[LANES - 1]

    cnt = lax.fori_loop(0, N_GROUPS, cgroup, jnp.int32(0))

    for k in range(CHUNK // LANES):
        crow[pl.ds(cnt + k * LANES, LANES)] = jnp.zeros((LANES,), jnp.int32)
        cseg[pl.ds(cnt + k * LANES, LANES)] = jnp.full((LANES,), TRASH,
                                                       jnp.int32)

    n_chunks = (cnt + CHUNK - 1) // CHUNK

    def gather(coff, buf, sem):
        pltpu.async_copy(
            x_hbm.at[crow.at[pl.ds(coff, CHUNK)], pl.ds(cbase, COLS)],
            buf, sem)

    def gwait(buf, sem):
        pltpu.make_async_copy(
            x_hbm.at[crow.at[pl.ds(0, CHUNK)], pl.ds(cbase, COLS)],
            buf, sem).wait()

    gsems = (gsem0, gsem1, gsem2, gsem3, gsem4, gsem5)
    for b in range(NBUF - 1):
        @pl.when(jnp.int32(b) < n_chunks)
        def _(b=b):
            gather(pl.multiple_of(jnp.int32(b * CHUNK), 8),
                   xbuf.at[b], gsems[b])

    def chunk_body(ci, carry):
        par = lax.rem(ci, NBUF)
        noff = pl.multiple_of((ci + NBUF - 1) * CHUNK, 8)

        for b in range(NBUF):
            @pl.when(par == b)
            def _(b=b):
                gwait(xbuf.at[b], gsems[b])
                nb = (b + NBUF - 1) % NBUF

                @pl.when(ci + NBUF - 1 < n_chunks)
                def _(b=b, nb=nb):
                    gather(noff, xbuf.at[nb], gsems[nb])

        coff = pl.multiple_of(ci * CHUNK, 8)

        def group_body(gi, carry):
            prev = carry[0]
            seg16 = cseg[pl.ds(coff + gi * LANES, LANES)]
            one_run = jnp.all(seg16 == prev)

            def fast(carry):
                prev, *acc = carry
                for r2 in range(LANES):
                    acc = [acc[k] + xbuf[par, gi * LANES + r2,
                                         pl.ds(k * LANES, LANES)]
                           for k in range(KVECS)]
                return (prev, *acc)

            def slow(carry):
                prev, *acc = carry
                for r2 in range(LANES):
                    seg = seg16[r2]
                    same = seg == prev
                    new_acc = []
                    for k in range(KVECS):
                        accum[prev, pl.ds(k * LANES, LANES)] = acc[k]
                        a = jnp.where(same, acc[k], zero16)
                        a = a + xbuf[par, gi * LANES + r2,
                                     pl.ds(k * LANES, LANES)]
                        accum[seg, pl.ds(k * LANES, LANES)] = a
                        new_acc.append(a)
                    acc = new_acc
                    prev = seg
                return (prev, *acc)

            return lax.cond(one_run, fast, slow, carry)

        return lax.fori_loop(0, CHUNK // LANES, group_body, carry)

    carry0 = (jnp.int32(TRASH),) + (zero16,) * KVECS
    fprev, *facc = lax.fori_loop(0, n_chunks, chunk_body, carry0)
    for k in range(KVECS):
        accum[fprev, pl.ds(k * LANES, LANES)] = facc[k]

    pltpu.sync_copy(accum.at[pl.ds(0, N_GRAPHS)], part_hbm.at[c].at[s])
    plsc.subcore_barrier()

    def stripe(t):
        return part_hbm.at[c].at[t].at[pl.ds(s * OUT_STRIPE, OUT_STRIPE)]

    pltpu.async_copy(stripe(jnp.int32(0)), tmp.at[0], rsem0)

    def tsum(t, carry):
        par = lax.rem(t, 2)

        @pl.when(par == 0)
        def _():
            pltpu.make_async_copy(stripe(t), tmp.at[0], rsem0).wait()

            @pl.when(t + 1 < N_SUBCORES)
            def _():
                pltpu.async_copy(stripe(t + 1), tmp.at[1], rsem1)

        @pl.when(par == 1)
        def _():
            pltpu.make_async_copy(stripe(t), tmp.at[1], rsem1).wait()

            @pl.when(t + 1 < N_SUBCORES)
            def _():
                pltpu.async_copy(stripe(t + 1), tmp.at[0], rsem0)

        def radd(r, carry2):
            for k in range(KVECS):
                sl = pl.ds(k * LANES, LANES)
                v = tmp[par, r, sl]
                rbuf[r, sl] = jnp.where(t == 0, v, rbuf[r, sl] + v)
            return carry2

        lax.fori_loop(0, OUT_STRIPE, radd, 0)
        return carry

    lax.fori_loop(0, N_SUBCORES, tsum, 0)

    pltpu.sync_copy(rbuf, out_hbm.at[pl.ds(s * OUT_STRIPE, OUT_STRIPE),
                                     pl.ds(cbase, COLS)])


_mesh = plsc.VectorSubcoreMesh(core_axis_name="c", subcore_axis_name="s",
                               num_cores=N_CORES, num_subcores=N_SUBCORES)

_sc_call = pl.kernel(
    _sc_body,
    out_type=(jax.ShapeDtypeStruct((N_GRAPHS, D_FEAT), jnp.float32),
              jax.ShapeDtypeStruct((N_CORES, N_SUBCORES, N_GRAPHS, COLS),
                                   jnp.float32)),
    mesh=_mesh,
    compiler_params=pltpu.CompilerParams(needs_layout_passes=False),
    scratch_types=[
        pltpu.VMEM((IDXBUF,), jnp.int32),
        pltpu.VMEM((IDXBUF,), jnp.int32),
        pltpu.VMEM((CBUF,), jnp.int32),
        pltpu.VMEM((CBUF,), jnp.int32),
        pltpu.VMEM((NBUF, CHUNK, COLS), jnp.float32),
        pltpu.VMEM((ACC_ROWS, COLS), jnp.float32),
        pltpu.VMEM((2, OUT_STRIPE, COLS), jnp.float32),
        pltpu.VMEM((OUT_STRIPE, COLS), jnp.float32),
        pltpu.SemaphoreType.DMA,
        pltpu.SemaphoreType.DMA,
        pltpu.SemaphoreType.DMA,
        pltpu.SemaphoreType.DMA,
        pltpu.SemaphoreType.DMA,
        pltpu.SemaphoreType.DMA,
        pltpu.SemaphoreType.DMA,
        pltpu.SemaphoreType.DMA,
    ],
)


@jax.jit
def kernel(x, atom_origin_type, batch):
    t = atom_origin_type.astype(jnp.int32)
    b = batch.astype(jnp.int32)
    out, _ = _sc_call(x, t, b)
    return out


# speedup vs baseline: 5.3027x; 1.0388x over previous
import jax
import jax.numpy as jnp
from jax import lax
from jax.experimental import pallas as pl
from jax.experimental.pallas import tpu as pltpu
from jax.experimental.pallas import tpu_sc as plsc

N_NODES = 100000
D_FEAT = 256
N_GRAPHS = 512
N_CORES = 2
N_SUBCORES = 16
COLS = D_FEAT // N_CORES
ROWS_PER_T = N_NODES // N_SUBCORES
LANES = 16
KVECS = COLS // LANES
N_GROUPS = (ROWS_PER_T + LANES - 1) // LANES
CHUNK = 16
NBUF = 8
TRASH = N_GRAPHS
ACC_ROWS = 513
OUT_STRIPE = N_GRAPHS // N_SUBCORES
STAGE = ROWS_PER_T + 6
IDXBUF = STAGE + 10
CBUF = ROWS_PER_T + CHUNK + 22


def _sc_body(x_hbm, type_hbm, batch_hbm, out_hbm, part_hbm,
             type_v, batch_v, crow, cseg, xbuf, accum, tmp, rbuf,
             gsem0, gsem1, gsem2, gsem3, gsem4, gsem5, gsem6, gsem7, rsem0, rsem1):
    c = lax.axis_index("c")
    s = lax.axis_index("s")
    rbase = s * ROWS_PER_T
    shift = lax.rem(rbase, 8)
    abase = pl.multiple_of(rbase - shift, 8)
    cbase = pl.multiple_of(c * COLS, COLS)

    zero16 = jnp.zeros((LANES,), jnp.float32)
    iota16 = lax.iota(jnp.int32, LANES)

    def zacc(r, carry):
        for k in range(KVECS):
            accum[r, pl.ds(k * LANES, LANES)] = zero16
        return carry

    lax.fori_loop(0, ACC_ROWS, zacc, 0)

    pltpu.sync_copy(type_hbm.at[pl.ds(abase, STAGE)],
                    type_v.at[pl.ds(0, STAGE)])
    pltpu.sync_copy(batch_hbm.at[pl.ds(abase, STAGE)],
                    batch_v.at[pl.ds(0, STAGE)])

    def cgroup(gi, cnt):
        o = gi * LANES + shift
        t16 = type_v[pl.ds(o, LANES)]
        seg16 = batch_v[pl.ds(o, LANES)]
        nvalid = jnp.minimum(ROWS_PER_T - gi * LANES, LANES)
        mask = jnp.logical_and(t16 == 0, iota16 < nvalid)
        rid16 = rbase + gi * LANES + iota16
        mi = mask.astype(jnp.int32)
        incl = jnp.cumsum(mi)
        pos = jnp.where(mask, cnt + incl - mi, CBUF - 1)
        plsc.store_scatter(crow, [pos], rid16)
        plsc.store_scatter(cseg, [pos], seg16)
        return cnt + incl[LANES - 1]

    cnt = lax.fori_loop(0, N_GROUPS, cgroup, jnp.int32(0))

    for k in range(CHUNK // LANES):
        crow[pl.ds(cnt + k * LANES, LANES)] = jnp.zeros((LANES,), jnp.int32)
        cseg[pl.ds(cnt + k * LANES, LANES)] = jnp.full((LANES,), TRASH,
                                                       jnp.int32)

    n_chunks = (cnt + CHUNK - 1) // CHUNK

    def gather(coff, buf, sem):
        pltpu.async_copy(
            x_hbm.at[crow.at[pl.ds(coff, CHUNK)], pl.ds(cbase, COLS)],
            buf, sem)

    def gwait(buf, sem):
        pltpu.make_async_copy(
            x_hbm.at[crow.at[pl.ds(0, CHUNK)], pl.ds(cbase, COLS)],
            buf, sem).wait()

    gsems = (gsem0, gsem1, gsem2, gsem3, gsem4, gsem5, gsem6, gsem7)
    for b in range(NBUF - 1):
        @pl.when(jnp.int32(b) < n_chunks)
        def _(b=b):
            gather(pl.multiple_of(jnp.int32(b * CHUNK), 8),
                   xbuf.at[b], gsems[b])

    def chunk_body(ci, carry):
        par = lax.rem(ci, NBUF)
        noff = pl.multiple_of((ci + NBUF - 1) * CHUNK, 8)

        for b in range(NBUF):
            @pl.when(par == b)
            def _(b=b):
                gwait(xbuf.at[b], gsems[b])
                nb = (b + NBUF - 1) % NBUF

                @pl.when(ci + NBUF - 1 < n_chunks)
                def _(b=b, nb=nb):
                    gather(noff, xbuf.at[nb], gsems[nb])

        coff = pl.multiple_of(ci * CHUNK, 8)

        def group_body(gi, carry):
            prev = carry[0]
            seg16 = cseg[pl.ds(coff + gi * LANES, LANES)]
            one_run = jnp.all(seg16 == prev)

            def fast(carry):
                prev, *acc = carry
                for r2 in range(LANES):
                    acc = [acc[k] + xbuf[par, gi * LANES + r2,
                                         pl.ds(k * LANES, LANES)]
                           for k in range(KVECS)]
                return (prev, *acc)

            def slow(carry):
                prev, *acc = carry
                for r2 in range(LANES):
                    seg = seg16[r2]
                    same = seg == prev
                    new_acc = []
                    for k in range(KVECS):
                        accum[prev, pl.ds(k * LANES, LANES)] = acc[k]
                        a = jnp.where(same, acc[k], zero16)
                        a = a + xbuf[par, gi * LANES + r2,
                                     pl.ds(k * LANES, LANES)]
                        accum[seg, pl.ds(k * LANES, LANES)] = a
                        new_acc.append(a)
                    acc = new_acc
                    prev = seg
                return (prev, *acc)

            return lax.cond(one_run, fast, slow, carry)

        return lax.fori_loop(0, CHUNK // LANES, group_body, carry)

    carry0 = (jnp.int32(TRASH),) + (zero16,) * KVECS
    fprev, *facc = lax.fori_loop(0, n_chunks, chunk_body, carry0)
    for k in range(KVECS):
        accum[fprev, pl.ds(k * LANES, LANES)] = facc[k]

    pltpu.sync_copy(accum.at[pl.ds(0, N_GRAPHS)], part_hbm.at[c].at[s])
    plsc.subcore_barrier()

    def stripe(t):
        return part_hbm.at[c].at[t].at[pl.ds(s * OUT_STRIPE, OUT_STRIPE)]

    pltpu.async_copy(stripe(jnp.int32(0)), tmp.at[0], rsem0)

    def tsum(t, carry):
        par = lax.rem(t, 2)

        @pl.when(par == 0)
        def _():
            pltpu.make_async_copy(stripe(t), tmp.at[0], rsem0).wait()

            @pl.when(t + 1 < N_SUBCORES)
            def _():
                pltpu.async_copy(stripe(t + 1), tmp.at[1], rsem1)

        @pl.when(par == 1)
        def _():
            pltpu.make_async_copy(stripe(t), tmp.at[1], rsem1).wait()

            @pl.when(t + 1 < N_SUBCORES)
            def _():
                pltpu.async_copy(stripe(t + 1), tmp.at[0], rsem0)

        def radd(r, carry2):
            for k in range(KVECS):
                sl = pl.ds(k * LANES, LANES)
                v = tmp[par, r, sl]
                rbuf[r, sl] = jnp.where(t == 0, v, rbuf[r, sl] + v)
            return carry2

        lax.fori_loop(0, OUT_STRIPE, radd, 0)
        return carry

    lax.fori_loop(0, N_SUBCORES, tsum, 0)

    pltpu.sync_copy(rbuf, out_hbm.at[pl.ds(s * OUT_STRIPE, OUT_STRIPE),
                                     pl.ds(cbase, COLS)])


_mesh = plsc.VectorSubcoreMesh(core_axis_name="c", subcore_axis_name="s",
                               num_cores=N_CORES, num_subcores=N_SUBCORES)

_sc_call = pl.kernel(
    _sc_body,
    out_type=(jax.ShapeDtypeStruct((N_GRAPHS, D_FEAT), jnp.float32),
              jax.ShapeDtypeStruct((N_CORES, N_SUBCORES, N_GRAPHS, COLS),
                                   jnp.float32)),
    mesh=_mesh,
    compiler_params=pltpu.CompilerParams(needs_layout_passes=False),
    scratch_types=[
        pltpu.VMEM((IDXBUF,), jnp.int32),
        pltpu.VMEM((IDXBUF,), jnp.int32),
        pltpu.VMEM((CBUF,), jnp.int32),
        pltpu.VMEM((CBUF,), jnp.int32),
        pltpu.VMEM((NBUF, CHUNK, COLS), jnp.float32),
        pltpu.VMEM((ACC_ROWS, COLS), jnp.float32),
        pltpu.VMEM((2, OUT_STRIPE, COLS), jnp.float32),
        pltpu.VMEM((OUT_STRIPE, COLS), jnp.float32),
        pltpu.SemaphoreType.DMA,
        pltpu.SemaphoreType.DMA,
        pltpu.SemaphoreType.DMA,
        pltpu.SemaphoreType.DMA,
        pltpu.SemaphoreType.DMA,
        pltpu.SemaphoreType.DMA,
        pltpu.SemaphoreType.DMA,
        pltpu.SemaphoreType.DMA,
        pltpu.SemaphoreType.DMA,
        pltpu.SemaphoreType.DMA,
    ],
)


@jax.jit
def kernel(x, atom_origin_type, batch):
    t = atom_origin_type.astype(jnp.int32)
    b = batch.astype(jnp.int32)
    out, _ = _sc_call(x, t, b)
    return out


# speedup vs baseline: 5.3851x; 1.0155x over previous
import jax
import jax.numpy as jnp
from jax import lax
from jax.experimental import pallas as pl
from jax.experimental.pallas import tpu as pltpu
from jax.experimental.pallas import tpu_sc as plsc

N_NODES = 100000
D_FEAT = 256
N_GRAPHS = 512
N_CORES = 2
N_SUBCORES = 16
COLS = D_FEAT // N_CORES
ROWS_PER_T = N_NODES // N_SUBCORES
LANES = 16
KVECS = COLS // LANES
N_GROUPS = (ROWS_PER_T + LANES - 1) // LANES
CHUNK = 16
NBUF = 12
TRASH = N_GRAPHS
ACC_ROWS = 513
OUT_STRIPE = N_GRAPHS // N_SUBCORES
STAGE = ROWS_PER_T + 6
IDXBUF = STAGE + 10
CBUF = ROWS_PER_T + CHUNK + 22


def _sc_body(x_hbm, type_hbm, batch_hbm, out_hbm, part_hbm,
             type_v, batch_v, crow, cseg, xbuf, accum, tmp, rbuf,
             gsem0, gsem1, gsem2, gsem3, gsem4, gsem5, gsem6, gsem7,
             gsem8, gsem9, gsem10, gsem11, rsem0, rsem1):
    c = lax.axis_index("c")
    s = lax.axis_index("s")
    rbase = s * ROWS_PER_T
    shift = lax.rem(rbase, 8)
    abase = pl.multiple_of(rbase - shift, 8)
    cbase = pl.multiple_of(c * COLS, COLS)

    zero16 = jnp.zeros((LANES,), jnp.float32)
    iota16 = lax.iota(jnp.int32, LANES)

    def zacc(r, carry):
        for k in range(KVECS):
            accum[r, pl.ds(k * LANES, LANES)] = zero16
        return carry

    lax.fori_loop(0, ACC_ROWS, zacc, 0)

    pltpu.sync_copy(type_hbm.at[pl.ds(abase, STAGE)],
                    type_v.at[pl.ds(0, STAGE)])
    pltpu.sync_copy(batch_hbm.at[pl.ds(abase, STAGE)],
                    batch_v.at[pl.ds(0, STAGE)])

    def cgroup(gi, cnt):
        o = gi * LANES + shift
        t16 = type_v[pl.ds(o, LANES)]
        seg16 = batch_v[pl.ds(o, LANES)]
        nvalid = jnp.minimum(ROWS_PER_T - gi * LANES, LANES)
        mask = jnp.logical_and(t16 == 0, iota16 < nvalid)
        rid16 = rbase + gi * LANES + iota16
        mi = mask.astype(jnp.int32)
        incl = jnp.cumsum(mi)
        pos = jnp.where(mask, cnt + incl - mi, CBUF - 1)
        plsc.store_scatter(crow, [pos], rid16)
        plsc.store_scatter(cseg, [pos], seg16)
        return cnt + incl[LANES - 1]

    cnt = lax.fori_loop(0, N_GROUPS, cgroup, jnp.int32(0))

    for k in range(CHUNK // LANES):
        crow[pl.ds(cnt + k * LANES, LANES)] = jnp.zeros((LANES,), jnp.int32)
        cseg[pl.ds(cnt + k * LANES, LANES)] = jnp.full((LANES,), TRASH,
                                                       jnp.int32)

    n_chunks = (cnt + CHUNK - 1) // CHUNK

    def gather(coff, buf, sem):
        pltpu.async_copy(
            x_hbm.at[crow.at[pl.ds(coff, CHUNK)], pl.ds(cbase, COLS)],
            buf, sem)

    def gwait(buf, sem):
        pltpu.make_async_copy(
            x_hbm.at[crow.at[pl.ds(0, CHUNK)], pl.ds(cbase, COLS)],
            buf, sem).wait()

    gsems = (gsem0, gsem1, gsem2, gsem3, gsem4, gsem5, gsem6, gsem7,
             gsem8, gsem9, gsem10, gsem11)
    for b in range(NBUF - 1):
        @pl.when(jnp.int32(b) < n_chunks)
        def _(b=b):
            gather(pl.multiple_of(jnp.int32(b * CHUNK), 8),
                   xbuf.at[b], gsems[b])

    def chunk_body(ci, carry):
        par = lax.rem(ci, NBUF)
        noff = pl.multiple_of((ci + NBUF - 1) * CHUNK, 8)

        for b in range(NBUF):
            @pl.when(par == b)
            def _(b=b):
                gwait(xbuf.at[b], gsems[b])
                nb = (b + NBUF - 1) % NBUF

                @pl.when(ci + NBUF - 1 < n_chunks)
                def _(b=b, nb=nb):
                    gather(noff, xbuf.at[nb], gsems[nb])

        coff = pl.multiple_of(ci * CHUNK, 8)

        def group_body(gi, carry):
            prev = carry[0]
            seg16 = cseg[pl.ds(coff + gi * LANES, LANES)]
            one_run = jnp.all(seg16 == prev)

            def fast(carry):
                prev, *acc = carry
                for r2 in range(LANES):
                    acc = [acc[k] + xbuf[par, gi * LANES + r2,
                                         pl.ds(k * LANES, LANES)]
                           for k in range(KVECS)]
                return (prev, *acc)

            def slow(carry):
                prev, *acc = carry
                for r2 in range(LANES):
                    seg = seg16[r2]
                    same = seg == prev
                    new_acc = []
                    for k in range(KVECS):
                        accum[prev, pl.ds(k * LANES, LANES)] = acc[k]
                        a = jnp.where(same, acc[k], zero16)
                        a = a + xbuf[par, gi * LANES + r2,
                                     pl.ds(k * LANES, LANES)]
                        accum[seg, pl.ds(k * LANES, LANES)] = a
                        new_acc.append(a)
                    acc = new_acc
                    prev = seg
                return (prev, *acc)

            return lax.cond(one_run, fast, slow, carry)

        return lax.fori_loop(0, CHUNK // LANES, group_body, carry)

    carry0 = (jnp.int32(TRASH),) + (zero16,) * KVECS
    fprev, *facc = lax.fori_loop(0, n_chunks, chunk_body, carry0)
    for k in range(KVECS):
        accum[fprev, pl.ds(k * LANES, LANES)] = facc[k]

    pltpu.sync_copy(accum.at[pl.ds(0, N_GRAPHS)], part_hbm.at[c].at[s])
    plsc.subcore_barrier()

    def stripe(t):
        return part_hbm.at[c].at[t].at[pl.ds(s * OUT_STRIPE, OUT_STRIPE)]

    pltpu.async_copy(stripe(jnp.int32(0)), tmp.at[0], rsem0)

    def tsum(t, carry):
        par = lax.rem(t, 2)

        @pl.when(par == 0)
        def _():
            pltpu.make_async_copy(stripe(t), tmp.at[0], rsem0).wait()

            @pl.when(t + 1 < N_SUBCORES)
            def _():
                pltpu.async_copy(stripe(t + 1), tmp.at[1], rsem1)

        @pl.when(par == 1)
        def _():
            pltpu.make_async_copy(stripe(t), tmp.at[1], rsem1).wait()

            @pl.when(t + 1 < N_SUBCORES)
            def _():
                pltpu.async_copy(stripe(t + 1), tmp.at[0], rsem0)

        def radd(r, carry2):
            for k in range(KVECS):
                sl = pl.ds(k * LANES, LANES)
                v = tmp[par, r, sl]
                rbuf[r, sl] = jnp.where(t == 0, v, rbuf[r, sl] + v)
            return carry2

        lax.fori_loop(0, OUT_STRIPE, radd, 0)
        return carry

    lax.fori_loop(0, N_SUBCORES, tsum, 0)

    pltpu.sync_copy(rbuf, out_hbm.at[pl.ds(s * OUT_STRIPE, OUT_STRIPE),
                                     pl.ds(cbase, COLS)])


_mesh = plsc.VectorSubcoreMesh(core_axis_name="c", subcore_axis_name="s",
                               num_cores=N_CORES, num_subcores=N_SUBCORES)

_sc_call = pl.kernel(
    _sc_body,
    out_type=(jax.ShapeDtypeStruct((N_GRAPHS, D_FEAT), jnp.float32),
              jax.ShapeDtypeStruct((N_CORES, N_SUBCORES, N_GRAPHS, COLS),
                                   jnp.float32)),
    mesh=_mesh,
    compiler_params=pltpu.CompilerParams(needs_layout_passes=False),
    scratch_types=[
        pltpu.VMEM((IDXBUF,), jnp.int32),
        pltpu.VMEM((IDXBUF,), jnp.int32),
        pltpu.VMEM((CBUF,), jnp.int32),
        pltpu.VMEM((CBUF,), jnp.int32),
        pltpu.VMEM((NBUF, CHUNK, COLS), jnp.float32),
        pltpu.VMEM((ACC_ROWS, COLS), jnp.float32),
        pltpu.VMEM((2, OUT_STRIPE, COLS), jnp.float32),
        pltpu.VMEM((OUT_STRIPE, COLS), jnp.float32),
        pltpu.SemaphoreType.DMA,
        pltpu.SemaphoreType.DMA,
        pltpu.SemaphoreType.DMA,
        pltpu.SemaphoreType.DMA,
        pltpu.SemaphoreType.DMA,
        pltpu.SemaphoreType.DMA,
        pltpu.SemaphoreType.DMA,
        pltpu.SemaphoreType.DMA,
        pltpu.SemaphoreType.DMA,
        pltpu.SemaphoreType.DMA,
        pltpu.SemaphoreType.DMA,
        pltpu.SemaphoreType.DMA,
        pltpu.SemaphoreType.DMA,
        pltpu.SemaphoreType.DMA,
    ],
)


@jax.jit
def kernel(x, atom_origin_type, batch):
    t = atom_origin_type.astype(jnp.int32)
    b = batch.astype(jnp.int32)
    out, _ = _sc_call(x, t, b)
    return out
